# baseline (speedup 1.0000x reference)
import functools
import jax
import jax.numpy as jnp
from jax import lax
from jax.experimental import pallas as pl
from jax.experimental.pallas import tpu as pltpu
from jax.experimental.pallas import tpu_sc as plsc

NC = 2
NS = 16
NW = NC * NS
K = 128


def _make_sc_agg(Np, D, C0, C1, with_count):
    R = Np // NS

    NBUF = 4
    LAG = NBUF // 2

    mesh = plsc.VectorSubcoreMesh(core_axis_name="c", subcore_axis_name="s")
    out_type = [jax.ShapeDtypeStruct((NC, Np, D), jnp.bfloat16)]
    scratch = (
        [pltpu.VMEM((C0, K), jnp.int32),
         pltpu.VMEM((C0, K), jnp.int32)] +
        [pltpu.VMEM((K, D), jnp.bfloat16)] * NBUF +
        [pltpu.VMEM_SHARED((Np, D), jnp.bfloat16)] +
        [pltpu.SemaphoreType.DMA] * (2 * NBUF)
    )
    if with_count:
        out_type.append(jax.ShapeDtypeStruct((Np,), jnp.float32))
        out_type.append(jax.ShapeDtypeStruct((Np,), jnp.float32))
        scratch += [
            pltpu.VMEM((Np,), jnp.float32),
            pltpu.VMEM((R,), jnp.float32),
            pltpu.VMEM((R,), jnp.float32),
            pltpu.VMEM_SHARED((NS, Np), jnp.float32),
        ]

    def body(src_hbm, dst_hbm, y_hbm, zeros_hbm, *rest):
        if with_count:
            (zeros1_hbm, out_hbm, cnt0_hbm, cnt1_hbm, src_v, dst_v) = rest[:6]
            bufs = rest[6:6 + NBUF]
            acc_sh = rest[6 + NBUF]
            gsems = rest[7 + NBUF:7 + 2 * NBUF]
            ssems = rest[7 + 2 * NBUF:7 + 3 * NBUF]
            hist_v, tmp_v, csum_v, stage_sh = rest[7 + 3 * NBUF:]
        else:
            (out_hbm, src_v, dst_v) = rest[:3]
            bufs = rest[3:3 + NBUF]
            acc_sh = rest[3 + NBUF]
            gsems = rest[4 + NBUF:4 + 2 * NBUF]
            ssems = rest[4 + 2 * NBUF:4 + 3 * NBUF]
        cid = lax.axis_index("c")
        sid = lax.axis_index("s")
        rows = pl.ds(sid * R, R)

        pltpu.sync_copy(zeros_hbm.at[rows], acc_sh.at[rows])
        if with_count:
            pltpu.sync_copy(zeros1_hbm, hist_v)
        plsc.subcore_barrier()

        ones16 = jnp.ones((16,), jnp.float32)

        def run_pipeline(C, base):
            pltpu.sync_copy(src_hbm.at[pl.ds(base, C)],
                            src_v.at[pl.ds(0, C)])
            pltpu.sync_copy(dst_hbm.at[pl.ds(base, C)],
                            dst_v.at[pl.ds(0, C)])

            for j in range(min(LAG, C)):
                pltpu.async_copy(y_hbm.at[src_v.at[j]], bufs[j], gsems[j])

            def step(c, _):
                if with_count:
                    for j in range(K // 16):
                        idx = dst_v[c, pl.ds(j * 16, 16)]
                        plsc.addupdate_scatter(hist_v, [idx], ones16)
                cur = lax.rem(c, NBUF)
                for b in range(NBUF):
                    @pl.when(cur == b)
                    def _():
                        pltpu.make_async_copy(y_hbm.at[src_v.at[c]],
                                              bufs[b], gsems[b]).wait()
                        pltpu.async_copy(bufs[b], acc_sh.at[dst_v.at[c]],
                                         ssems[b], add=True)
                        nb = (b + LAG) % NBUF

                        @pl.when(c + LAG < C)
                        def _():
                            @pl.when(c >= LAG)
                            def _():
                                pltpu.make_async_copy(
                                    bufs[nb], acc_sh.at[dst_v.at[c]],
                                    ssems[nb]).wait()

                            pltpu.async_copy(y_hbm.at[src_v.at[c + LAG]],
                                             bufs[nb], gsems[nb])
                return 0

            lax.fori_loop(0, C, step, 0)

            for j in range(max(0, C - NBUF), C):
                b = j % NBUF
                pltpu.make_async_copy(bufs[b], acc_sh.at[dst_v.at[j]],
                                      ssems[b]).wait()

        @pl.when(cid == 0)
        def _():
            run_pipeline(C0, sid * C0)

        @pl.when(cid == 1)
        def _():
            run_pipeline(C1, NS * C0 + sid * C1)

        plsc.subcore_barrier()

        pltpu.sync_copy(acc_sh.at[rows], out_hbm.at[cid, rows])
        if with_count:
            pltpu.sync_copy(hist_v, stage_sh.at[sid])
            plsc.subcore_barrier()
            pltpu.sync_copy(stage_sh.at[0, rows], csum_v)

            def accum(t, _):
                pltpu.sync_copy(stage_sh.at[t, rows], tmp_v)

                def addv(j, _):
                    sl = pl.ds(j * 16, 16)
                    csum_v[sl] = csum_v[sl] + tmp_v[sl]
                    return 0

                lax.fori_loop(0, R // 16, addv, 0)
                return 0

            lax.fori_loop(1, NS, accum, 0)

            @pl.when(cid == 0)
            def _():
                pltpu.sync_copy(csum_v, cnt0_hbm.at[rows])

            @pl.when(cid == 1)
            def _():
                pltpu.sync_copy(csum_v, cnt1_hbm.at[rows])

    return pl.kernel(body, out_type=out_type, mesh=mesh,
                     scratch_types=scratch,
                     compiler_params=pltpu.CompilerParams(
                         use_tc_tiling_on_sc=False,
                         needs_layout_passes=False))


def _mm2_body(x_ref, wl_ref, wr_ref, b_ref, y_ref, z_ref):
    x = x_ref[...]
    y = jnp.dot(x, wl_ref[...], preferred_element_type=jnp.float32)
    y_ref[...] = y.astype(jnp.bfloat16).reshape(y_ref.shape)
    z_ref[...] = (jnp.dot(x, wr_ref[...], preferred_element_type=jnp.float32)
                  + b_ref[...])


def _mean_relu(p0_ref, p1_ref, c0_ref, c1_ref, z_ref):
    B, D = z_ref.shape
    recip = 1.0 / jnp.maximum(c0_ref[...] + c1_ref[...], 1.0)
    psum = (p0_ref[...].reshape(B, D).astype(jnp.float32)
            + p1_ref[...].reshape(B, D).astype(jnp.float32))
    return jnp.maximum(psum * recip + z_ref[...], 0.0)


def _mid_body(p0_ref, p1_ref, c0_ref, c1_ref, z_ref, wl_ref, wr_ref, b_ref,
              y_ref, z2_ref):
    h = _mean_relu(p0_ref, p1_ref, c0_ref, c1_ref, z_ref)
    y = jnp.dot(h, wl_ref[...], preferred_element_type=jnp.float32)
    y_ref[...] = y.astype(jnp.bfloat16).reshape(y_ref.shape)
    z2_ref[...] = (jnp.dot(h, wr_ref[...], preferred_element_type=jnp.float32)
                   + b_ref[...])


def _fin_body(p0_ref, p1_ref, c0_ref, c1_ref, z_ref, out_ref):
    out_ref[...] = _mean_relu(p0_ref, p1_ref, c0_ref, c1_ref, z_ref)


def _row_spec(B, D):
    return pl.BlockSpec((B, D), lambda i: (i, 0))


def _pair_spec(B, D):
    return pl.BlockSpec((NC, B, D), lambda i: (0, i, 0))


def _full_spec(shape):
    return pl.BlockSpec(shape, lambda i: tuple(0 for _ in shape))


@jax.jit
def kernel(x, edge_index, W1_l, W1_r, b1, W2_l, W2_r, b2):
    N, D = x.shape
    E = edge_index.shape[1]
    Np = ((N + NS * 16 - 1) // (NS * 16)) * NS * 16
    CT = -(-E // (NS * K)) * NS
    Ep = CT * K
    CPP = CT // NS
    C0 = (CPP * 62 + 50) // 100
    C1 = CPP - C0
    B = 1024

    src = jnp.concatenate(
        [edge_index[0], jnp.zeros((Ep - E,), jnp.int32)])
    dst = jnp.concatenate(
        [edge_index[1], jnp.full((Ep - E,), N, jnp.int32)])
    src_r = src.reshape(CT, K)
    dst_r = dst.reshape(CT, K)
    zeros = jnp.zeros((Np, D), jnp.bfloat16)
    zeros1 = jnp.zeros((Np,), jnp.float32)
    b1r = b1.reshape(1, D)
    b2r = b2.reshape(1, D)
    xp = jnp.zeros((Np, D), jnp.float32).at[:N].set(x)

    grid = Np // B
    flat_bf16 = jax.ShapeDtypeStruct((Np * D,), jnp.bfloat16)
    fspec = pl.BlockSpec((B * D,), lambda i: (i,))

    def _plane_spec(plane):
        return pl.BlockSpec((B * D,), lambda i, p=plane: (i + p * grid,))

    mm2 = pl.pallas_call(
        _mm2_body, grid=(grid,),
        in_specs=[_row_spec(B, D), _full_spec((D, D)), _full_spec((D, D)),
                  _full_spec((1, D))],
        out_specs=[fspec, _row_spec(B, D)],
        out_shape=[flat_bf16,
                   jax.ShapeDtypeStruct((Np, D), jnp.float32)],
    )
    mid = pl.pallas_call(
        _mid_body, grid=(grid,),
        in_specs=[_plane_spec(0), _plane_spec(1), _row_spec(B, 1),
                  _row_spec(B, 1), _row_spec(B, D),
                  _full_spec((D, D)), _full_spec((D, D)), _full_spec((1, D))],
        out_specs=[fspec, _row_spec(B, D)],
        out_shape=[flat_bf16,
                   jax.ShapeDtypeStruct((Np, D), jnp.float32)],
    )
    fin = pl.pallas_call(
        _fin_body, grid=(grid,),
        in_specs=[_plane_spec(0), _plane_spec(1), _row_spec(B, 1),
                  _row_spec(B, 1), _row_spec(B, D)],
        out_specs=_row_spec(B, D),
        out_shape=jax.ShapeDtypeStruct((Np, D), jnp.float32),
    )

    sc1 = _make_sc_agg(Np, D, C0, C1, with_count=True)
    sc2 = _make_sc_agg(Np, D, C0, C1, with_count=False)

    y1f, z1 = mm2(xp, W1_l, W1_r, b1r)
    p1, cnt0, cnt1 = sc1(src_r, dst_r, y1f.reshape(Np, D), zeros, zeros1)
    c0 = cnt0.reshape(Np, 1)
    c1 = cnt1.reshape(Np, 1)
    p1f = p1.reshape(-1)
    y2f, z2 = mid(p1f, p1f, c0, c1, z1, W2_l, W2_r, b2r)
    (p2,) = sc2(src_r, dst_r, y2f.reshape(Np, D), zeros)
    p2f = p2.reshape(-1)
    return fin(p2f, p2f, c0, c1, z2)[:N]


# speedup vs baseline: 1.0474x; 1.0474x over previous
import functools
import jax
import jax.numpy as jnp
from jax import lax
from jax.experimental import pallas as pl
from jax.experimental.pallas import tpu as pltpu
from jax.experimental.pallas import tpu_sc as plsc

NC = 2
NS = 16
NW = NC * NS
K = 128


def _make_sc_agg(Np, D, C0, C1, with_count):
    R = Np // NS

    NBUF = 4
    LAG = NBUF // 2

    mesh = plsc.VectorSubcoreMesh(core_axis_name="c", subcore_axis_name="s")
    out_type = [jax.ShapeDtypeStruct((NC, Np, D), jnp.bfloat16)]
    scratch = (
        [pltpu.VMEM((C0, K), jnp.int32),
         pltpu.VMEM((C0, K), jnp.int32)] +
        [pltpu.VMEM((K, D), jnp.bfloat16)] * NBUF +
        [pltpu.VMEM_SHARED((Np, D), jnp.bfloat16)] +
        [pltpu.SemaphoreType.DMA] * (2 * NBUF)
    )
    if with_count:
        out_type.append(jax.ShapeDtypeStruct((Np,), jnp.float32))
        out_type.append(jax.ShapeDtypeStruct((Np,), jnp.float32))
        scratch += [
            pltpu.VMEM((Np,), jnp.float32),
            pltpu.VMEM((R,), jnp.float32),
            pltpu.VMEM((R,), jnp.float32),
            pltpu.VMEM_SHARED((NS, Np), jnp.float32),
        ]

    def body(src_hbm, dst_hbm, y_hbm, zeros_hbm, *rest):
        if with_count:
            (zeros1_hbm, out_hbm, cnt0_hbm, cnt1_hbm, src_v, dst_v) = rest[:6]
            bufs = rest[6:6 + NBUF]
            acc_sh = rest[6 + NBUF]
            gsems = rest[7 + NBUF:7 + 2 * NBUF]
            ssems = rest[7 + 2 * NBUF:7 + 3 * NBUF]
            hist_v, tmp_v, csum_v, stage_sh = rest[7 + 3 * NBUF:]
        else:
            (out_hbm, src_v, dst_v) = rest[:3]
            bufs = rest[3:3 + NBUF]
            acc_sh = rest[3 + NBUF]
            gsems = rest[4 + NBUF:4 + 2 * NBUF]
            ssems = rest[4 + 2 * NBUF:4 + 3 * NBUF]
        cid = lax.axis_index("c")
        sid = lax.axis_index("s")
        rows = pl.ds(sid * R, R)

        pltpu.sync_copy(zeros_hbm.at[rows], acc_sh.at[rows])
        if with_count:
            pltpu.sync_copy(zeros1_hbm, hist_v)
        plsc.subcore_barrier()

        ones16 = jnp.ones((16,), jnp.float32)

        def run_pipeline(C, base):
            pltpu.sync_copy(src_hbm.at[pl.ds(base, C)],
                            src_v.at[pl.ds(0, C)])
            pltpu.sync_copy(dst_hbm.at[pl.ds(base, C)],
                            dst_v.at[pl.ds(0, C)])

            for j in range(min(LAG, C)):
                pltpu.async_copy(y_hbm.at[src_v.at[j]], bufs[j], gsems[j])

            def step(c, _):
                if with_count:
                    for j in range(K // 16):
                        idx = dst_v[c, pl.ds(j * 16, 16)]
                        plsc.addupdate_scatter(hist_v, [idx], ones16)
                cur = lax.rem(c, NBUF)
                for b in range(NBUF):
                    @pl.when(cur == b)
                    def _():
                        pltpu.make_async_copy(y_hbm.at[src_v.at[c]],
                                              bufs[b], gsems[b]).wait()
                        pltpu.async_copy(bufs[b], acc_sh.at[dst_v.at[c]],
                                         ssems[b], add=True)
                        nb = (b + LAG) % NBUF

                        @pl.when(c + LAG < C)
                        def _():
                            @pl.when(c >= LAG)
                            def _():
                                pltpu.make_async_copy(
                                    bufs[nb], acc_sh.at[dst_v.at[c]],
                                    ssems[nb]).wait()

                            pltpu.async_copy(y_hbm.at[src_v.at[c + LAG]],
                                             bufs[nb], gsems[nb])
                return 0

            lax.fori_loop(0, C, step, 0)

            for j in range(max(0, C - NBUF), C):
                b = j % NBUF
                pltpu.make_async_copy(bufs[b], acc_sh.at[dst_v.at[j]],
                                      ssems[b]).wait()

        @pl.when(cid == 0)
        def _():
            run_pipeline(C0, sid * C0)

        @pl.when(cid == 1)
        def _():
            run_pipeline(C1, NS * C0 + sid * C1)

        plsc.subcore_barrier()

        pltpu.sync_copy(acc_sh.at[rows], out_hbm.at[cid, rows])
        if with_count:
            pltpu.sync_copy(hist_v, stage_sh.at[sid])
            plsc.subcore_barrier()
            pltpu.sync_copy(stage_sh.at[0, rows], csum_v)

            def accum(t, _):
                pltpu.sync_copy(stage_sh.at[t, rows], tmp_v)

                def addv(j, _):
                    sl = pl.ds(j * 16, 16)
                    csum_v[sl] = csum_v[sl] + tmp_v[sl]
                    return 0

                lax.fori_loop(0, R // 16, addv, 0)
                return 0

            lax.fori_loop(1, NS, accum, 0)

            @pl.when(cid == 0)
            def _():
                pltpu.sync_copy(csum_v, cnt0_hbm.at[rows])

            @pl.when(cid == 1)
            def _():
                pltpu.sync_copy(csum_v, cnt1_hbm.at[rows])

    return pl.kernel(body, out_type=out_type, mesh=mesh,
                     scratch_types=scratch,
                     compiler_params=pltpu.CompilerParams(
                         use_tc_tiling_on_sc=False,
                         needs_layout_passes=False))


def _mm2_body(x_ref, wl_ref, wr_ref, b_ref, y_ref, z_ref):
    x = x_ref[...]
    y = jnp.dot(x, wl_ref[...], preferred_element_type=jnp.float32)
    y_ref[...] = y.astype(jnp.bfloat16)
    z_ref[...] = (jnp.dot(x, wr_ref[...], preferred_element_type=jnp.float32)
                  + b_ref[...]).astype(jnp.bfloat16)


def _mean_relu(p_ref, c0_ref, c1_ref, z_ref):
    recip = 1.0 / jnp.maximum(c0_ref[...] + c1_ref[...], 1.0)
    psum = p_ref[0].astype(jnp.float32) + p_ref[1].astype(jnp.float32)
    return jnp.maximum(psum * recip + z_ref[...].astype(jnp.float32), 0.0)


def _mid_body(p_ref, c0_ref, c1_ref, z_ref, wl_ref, wr_ref, b_ref,
              y_ref, z2_ref):
    h = _mean_relu(p_ref, c0_ref, c1_ref, z_ref)
    y = jnp.dot(h, wl_ref[...], preferred_element_type=jnp.float32)
    y_ref[...] = y.astype(jnp.bfloat16)
    z2_ref[...] = (jnp.dot(h, wr_ref[...], preferred_element_type=jnp.float32)
                   + b_ref[...]).astype(jnp.bfloat16)


def _fin_body(p_ref, c0_ref, c1_ref, z_ref, out_ref):
    out_ref[...] = _mean_relu(p_ref, c0_ref, c1_ref, z_ref)


def _row_spec(B, D):
    return pl.BlockSpec((B, D), lambda i: (i, 0))


def _pair_spec(B, D):
    return pl.BlockSpec((NC, B, D), lambda i: (0, i, 0))


def _full_spec(shape):
    return pl.BlockSpec(shape, lambda i: tuple(0 for _ in shape))


@jax.jit
def kernel(x, edge_index, W1_l, W1_r, b1, W2_l, W2_r, b2):
    N, D = x.shape
    E = edge_index.shape[1]
    Np = ((N + NS * 16 - 1) // (NS * 16)) * NS * 16
    CT = -(-E // (NS * K)) * NS
    Ep = CT * K
    CPP = CT // NS
    C0 = (CPP * 62 + 50) // 100
    C1 = CPP - C0
    B = 2000

    src = jnp.concatenate(
        [edge_index[0], jnp.zeros((Ep - E,), jnp.int32)])
    dst = jnp.concatenate(
        [edge_index[1], jnp.full((Ep - E,), N, jnp.int32)])
    src_r = src.reshape(CT, K)
    dst_r = dst.reshape(CT, K)
    zeros = jnp.zeros((Np, D), jnp.bfloat16)
    zeros1 = jnp.zeros((Np,), jnp.float32)
    b1r = b1.reshape(1, D)
    b2r = b2.reshape(1, D)

    grid = N // B
    mm2 = pl.pallas_call(
        _mm2_body, grid=(grid,),
        in_specs=[_row_spec(B, D), _full_spec((D, D)), _full_spec((D, D)),
                  _full_spec((1, D))],
        out_specs=[_row_spec(B, D), _row_spec(B, D)],
        out_shape=[jax.ShapeDtypeStruct((N, D), jnp.bfloat16),
                   jax.ShapeDtypeStruct((N, D), jnp.bfloat16)],
    )
    mid = pl.pallas_call(
        _mid_body, grid=(grid,),
        in_specs=[_pair_spec(B, D), _row_spec(B, 1), _row_spec(B, 1),
                  _row_spec(B, D),
                  _full_spec((D, D)), _full_spec((D, D)), _full_spec((1, D))],
        out_specs=[_row_spec(B, D), _row_spec(B, D)],
        out_shape=[jax.ShapeDtypeStruct((N, D), jnp.bfloat16),
                   jax.ShapeDtypeStruct((N, D), jnp.bfloat16)],
    )
    fin = pl.pallas_call(
        _fin_body, grid=(grid,),
        in_specs=[_pair_spec(B, D), _row_spec(B, 1), _row_spec(B, 1),
                  _row_spec(B, D)],
        out_specs=_row_spec(B, D),
        out_shape=jax.ShapeDtypeStruct((N, D), jnp.float32),
    )

    sc1 = _make_sc_agg(Np, D, C0, C1, with_count=True)
    sc2 = _make_sc_agg(Np, D, C0, C1, with_count=False)

    y1, z1 = mm2(x, W1_l, W1_r, b1r)
    p1, cnt0, cnt1 = sc1(src_r, dst_r, y1, zeros, zeros1)
    c0 = cnt0.reshape(Np, 1)
    c1 = cnt1.reshape(Np, 1)
    y2, z2 = mid(p1, c0, c1, z1, W2_l, W2_r, b2r)
    (p2,) = sc2(src_r, dst_r, y2, zeros)
    return fin(p2, c0, c1, z2)


# speedup vs baseline: 1.0595x; 1.0116x over previous
import functools
import jax
import jax.numpy as jnp
from jax import lax
from jax.experimental import pallas as pl
from jax.experimental.pallas import tpu as pltpu
from jax.experimental.pallas import tpu_sc as plsc

NC = 2
NS = 16
NW = NC * NS
K = 128


def _make_sc_agg(Np, D, C0, C1, with_count):
    R = Np // NS

    NBUF = 4
    LAG = NBUF // 2

    mesh = plsc.VectorSubcoreMesh(core_axis_name="c", subcore_axis_name="s")
    out_type = [jax.ShapeDtypeStruct((NC, Np, D), jnp.bfloat16)]
    scratch = (
        [pltpu.VMEM((C0, K), jnp.int32),
         pltpu.VMEM((C0, K), jnp.int32)] +
        [pltpu.VMEM((K, D), jnp.bfloat16)] * NBUF +
        [pltpu.VMEM_SHARED((Np, D), jnp.bfloat16)] +
        [pltpu.SemaphoreType.DMA] * (2 * NBUF)
    )
    if with_count:
        out_type.append(jax.ShapeDtypeStruct((Np,), jnp.float32))
        out_type.append(jax.ShapeDtypeStruct((Np,), jnp.float32))
        scratch += [
            pltpu.VMEM((Np,), jnp.float32),
            pltpu.VMEM((R,), jnp.float32),
            pltpu.VMEM((R,), jnp.float32),
            pltpu.VMEM_SHARED((NS, Np), jnp.float32),
        ]

    def body(src_hbm, dst_hbm, y_hbm, zeros_hbm, *rest):
        if with_count:
            (zeros1_hbm, out_hbm, cnt0_hbm, cnt1_hbm, src_v, dst_v) = rest[:6]
            bufs = rest[6:6 + NBUF]
            acc_sh = rest[6 + NBUF]
            gsems = rest[7 + NBUF:7 + 2 * NBUF]
            ssems = rest[7 + 2 * NBUF:7 + 3 * NBUF]
            hist_v, tmp_v, csum_v, stage_sh = rest[7 + 3 * NBUF:]
        else:
            (out_hbm, src_v, dst_v) = rest[:3]
            bufs = rest[3:3 + NBUF]
            acc_sh = rest[3 + NBUF]
            gsems = rest[4 + NBUF:4 + 2 * NBUF]
            ssems = rest[4 + 2 * NBUF:4 + 3 * NBUF]
        cid = lax.axis_index("c")
        sid = lax.axis_index("s")
        rows = pl.ds(sid * R, R)

        pltpu.sync_copy(zeros_hbm.at[rows], acc_sh.at[rows])
        if with_count:
            pltpu.sync_copy(zeros1_hbm, hist_v)
        plsc.subcore_barrier()

        ones16 = jnp.ones((16,), jnp.float32)

        def run_pipeline(C, base):
            pltpu.sync_copy(src_hbm.at[pl.ds(base, C)],
                            src_v.at[pl.ds(0, C)])
            pltpu.sync_copy(dst_hbm.at[pl.ds(base, C)],
                            dst_v.at[pl.ds(0, C)])

            for j in range(min(LAG, C)):
                pltpu.async_copy(y_hbm.at[src_v.at[j]], bufs[j], gsems[j])

            def step(c, _):
                if with_count:
                    for j in range(K // 16):
                        idx = dst_v[c, pl.ds(j * 16, 16)]
                        plsc.addupdate_scatter(hist_v, [idx], ones16)
                cur = lax.rem(c, NBUF)
                for b in range(NBUF):
                    @pl.when(cur == b)
                    def _():
                        pltpu.make_async_copy(y_hbm.at[src_v.at[c]],
                                              bufs[b], gsems[b]).wait()
                        pltpu.async_copy(bufs[b], acc_sh.at[dst_v.at[c]],
                                         ssems[b], add=True)
                        nb = (b + LAG) % NBUF

                        @pl.when(c + LAG < C)
                        def _():
                            @pl.when(c >= LAG)
                            def _():
                                pltpu.make_async_copy(
                                    bufs[nb], acc_sh.at[dst_v.at[c]],
                                    ssems[nb]).wait()

                            pltpu.async_copy(y_hbm.at[src_v.at[c + LAG]],
                                             bufs[nb], gsems[nb])
                return 0

            lax.fori_loop(0, C, step, 0)

            for j in range(max(0, C - NBUF), C):
                b = j % NBUF
                pltpu.make_async_copy(bufs[b], acc_sh.at[dst_v.at[j]],
                                      ssems[b]).wait()

        @pl.when(cid == 0)
        def _():
            run_pipeline(C0, sid * C0)

        @pl.when(cid == 1)
        def _():
            run_pipeline(C1, NS * C0 + sid * C1)

        plsc.subcore_barrier()

        pltpu.sync_copy(acc_sh.at[rows], out_hbm.at[cid, rows])
        if with_count:
            pltpu.sync_copy(hist_v, stage_sh.at[sid])
            plsc.subcore_barrier()
            pltpu.sync_copy(stage_sh.at[0, rows], csum_v)

            def accum(t, _):
                pltpu.sync_copy(stage_sh.at[t, rows], tmp_v)

                def addv(j, _):
                    sl = pl.ds(j * 16, 16)
                    csum_v[sl] = csum_v[sl] + tmp_v[sl]
                    return 0

                lax.fori_loop(0, R // 16, addv, 0)
                return 0

            lax.fori_loop(1, NS, accum, 0)

            @pl.when(cid == 0)
            def _():
                pltpu.sync_copy(csum_v, cnt0_hbm.at[rows])

            @pl.when(cid == 1)
            def _():
                pltpu.sync_copy(csum_v, cnt1_hbm.at[rows])

    return pl.kernel(body, out_type=out_type, mesh=mesh,
                     scratch_types=scratch,
                     compiler_params=pltpu.CompilerParams(
                         use_tc_tiling_on_sc=False,
                         needs_layout_passes=False))


def _mm2_body(x_ref, wl_ref, wr_ref, b_ref, y_ref, z_ref):
    x = x_ref[...]
    y = jnp.dot(x, wl_ref[...], preferred_element_type=jnp.float32)
    y_ref[...] = y.astype(jnp.bfloat16)
    z_ref[...] = (jnp.dot(x, wr_ref[...], preferred_element_type=jnp.float32)
                  + b_ref[...]).astype(jnp.bfloat16)


def _mean_relu(p_ref, c0_ref, c1_ref, z_ref):
    B = z_ref.shape[0]
    i = pl.program_id(0)
    off = pl.multiple_of(i * B, 128)
    cnt = c0_ref[pl.ds(off, B)] + c1_ref[pl.ds(off, B)]
    recip = 1.0 / jnp.maximum(cnt, 1.0).reshape(B, 1)
    psum = p_ref[0].astype(jnp.float32) + p_ref[1].astype(jnp.float32)
    return jnp.maximum(psum * recip + z_ref[...].astype(jnp.float32), 0.0)


def _mid_body(p_ref, c0_ref, c1_ref, z_ref, wl_ref, wr_ref, b_ref,
              y_ref, z2_ref):
    h = _mean_relu(p_ref, c0_ref, c1_ref, z_ref)
    y = jnp.dot(h, wl_ref[...], preferred_element_type=jnp.float32)
    y_ref[...] = y.astype(jnp.bfloat16)
    z2_ref[...] = (jnp.dot(h, wr_ref[...], preferred_element_type=jnp.float32)
                   + b_ref[...]).astype(jnp.bfloat16)


def _fin_body(p_ref, c0_ref, c1_ref, z_ref, out_ref):
    out_ref[...] = _mean_relu(p_ref, c0_ref, c1_ref, z_ref)


def _row_spec(B, D):
    return pl.BlockSpec((B, D), lambda i: (i, 0))


def _vec_spec(Np):
    return pl.BlockSpec((Np,), lambda i: (0,))


def _pair_spec(B, D):
    return pl.BlockSpec((NC, B, D), lambda i: (0, i, 0))


def _full_spec(shape):
    return pl.BlockSpec(shape, lambda i: tuple(0 for _ in shape))


@jax.jit
def kernel(x, edge_index, W1_l, W1_r, b1, W2_l, W2_r, b2):
    N, D = x.shape
    E = edge_index.shape[1]
    Np = ((N + NS * 16 - 1) // (NS * 16)) * NS * 16
    CT = -(-E // (NS * K)) * NS
    Ep = CT * K
    CPP = CT // NS
    C0 = (CPP * 62 + 50) // 100
    C1 = CPP - C0
    B = 2048

    src = jnp.concatenate(
        [edge_index[0], jnp.zeros((Ep - E,), jnp.int32)])
    dst = jnp.concatenate(
        [edge_index[1], jnp.full((Ep - E,), N, jnp.int32)])
    src_r = src.reshape(CT, K)
    dst_r = dst.reshape(CT, K)
    zeros = jnp.zeros((Np, D), jnp.bfloat16)
    zeros1 = jnp.zeros((Np,), jnp.float32)
    b1r = b1.reshape(1, D)
    b2r = b2.reshape(1, D)

    grid = -(-N // B)
    mm2 = pl.pallas_call(
        _mm2_body, grid=(grid,),
        in_specs=[_row_spec(B, D), _full_spec((D, D)), _full_spec((D, D)),
                  _full_spec((1, D))],
        out_specs=[_row_spec(B, D), _row_spec(B, D)],
        out_shape=[jax.ShapeDtypeStruct((N, D), jnp.bfloat16),
                   jax.ShapeDtypeStruct((N, D), jnp.bfloat16)],
    )
    mid = pl.pallas_call(
        _mid_body, grid=(grid,),
        in_specs=[_pair_spec(B, D), _vec_spec(Np), _vec_spec(Np),
                  _row_spec(B, D),
                  _full_spec((D, D)), _full_spec((D, D)), _full_spec((1, D))],
        out_specs=[_row_spec(B, D), _row_spec(B, D)],
        out_shape=[jax.ShapeDtypeStruct((N, D), jnp.bfloat16),
                   jax.ShapeDtypeStruct((N, D), jnp.bfloat16)],
    )
    fin = pl.pallas_call(
        _fin_body, grid=(grid,),
        in_specs=[_pair_spec(B, D), _vec_spec(Np), _vec_spec(Np),
                  _row_spec(B, D)],
        out_specs=_row_spec(B, D),
        out_shape=jax.ShapeDtypeStruct((N, D), jnp.float32),
    )

    sc1 = _make_sc_agg(Np, D, C0, C1, with_count=True)
    sc2 = _make_sc_agg(Np, D, C0, C1, with_count=False)

    y1, z1 = mm2(x, W1_l, W1_r, b1r)
    p1, c0, c1 = sc1(src_r, dst_r, y1, zeros, zeros1)
    y2, z2 = mid(p1, c0, c1, z1, W2_l, W2_r, b2r)
    (p2,) = sc2(src_r, dst_r, y2, zeros)
    return fin(p2, c0, c1, z2)


# speedup vs baseline: 1.0716x; 1.0114x over previous
import functools
import jax
import jax.numpy as jnp
from jax import lax
from jax.experimental import pallas as pl
from jax.experimental.pallas import tpu as pltpu
from jax.experimental.pallas import tpu_sc as plsc

NC = 2
NS = 16
NW = NC * NS
K = 128


def _make_sc_agg(Np, D, C0, C1, with_count):
    R = Np // NS

    NBUF = 4
    LAG = NBUF // 2

    mesh = plsc.VectorSubcoreMesh(core_axis_name="c", subcore_axis_name="s")
    out_type = [jax.ShapeDtypeStruct((NC, Np, D), jnp.bfloat16)]
    scratch = (
        [pltpu.VMEM((C0, K), jnp.int32),
         pltpu.VMEM((C0, K), jnp.int32)] +
        [pltpu.VMEM((K, D), jnp.bfloat16)] * NBUF +
        [pltpu.VMEM_SHARED((Np, D), jnp.bfloat16)] +
        [pltpu.SemaphoreType.DMA] * (2 * NBUF)
    )
    if with_count:
        out_type.append(jax.ShapeDtypeStruct((Np,), jnp.float32))
        out_type.append(jax.ShapeDtypeStruct((Np,), jnp.float32))
        scratch += [
            pltpu.VMEM((Np,), jnp.float32),
            pltpu.VMEM((R,), jnp.float32),
            pltpu.VMEM((R,), jnp.float32),
            pltpu.VMEM_SHARED((NS, Np), jnp.float32),
        ]

    def body(src_hbm, dst_hbm, y_hbm, zeros_hbm, *rest):
        if with_count:
            (zeros1_hbm, out_hbm, cnt0_hbm, cnt1_hbm, src_v, dst_v) = rest[:6]
            bufs = rest[6:6 + NBUF]
            acc_sh = rest[6 + NBUF]
            gsems = rest[7 + NBUF:7 + 2 * NBUF]
            ssems = rest[7 + 2 * NBUF:7 + 3 * NBUF]
            hist_v, tmp_v, csum_v, stage_sh = rest[7 + 3 * NBUF:]
        else:
            (out_hbm, src_v, dst_v) = rest[:3]
            bufs = rest[3:3 + NBUF]
            acc_sh = rest[3 + NBUF]
            gsems = rest[4 + NBUF:4 + 2 * NBUF]
            ssems = rest[4 + 2 * NBUF:4 + 3 * NBUF]
        cid = lax.axis_index("c")
        sid = lax.axis_index("s")
        rows = pl.ds(sid * R, R)

        pltpu.sync_copy(zeros_hbm.at[rows], acc_sh.at[rows])
        if with_count:
            pltpu.sync_copy(zeros1_hbm, hist_v)
        plsc.subcore_barrier()

        ones16 = jnp.ones((16,), jnp.float32)

        def run_pipeline(C, base):
            pltpu.sync_copy(src_hbm.at[pl.ds(base, C)],
                            src_v.at[pl.ds(0, C)])
            pltpu.sync_copy(dst_hbm.at[pl.ds(base, C)],
                            dst_v.at[pl.ds(0, C)])

            for j in range(min(LAG, C)):
                pltpu.async_copy(y_hbm.at[src_v.at[j]], bufs[j], gsems[j])

            def step(c, _):
                if with_count:
                    for j in range(K // 16):
                        idx = dst_v[c, pl.ds(j * 16, 16)]
                        plsc.addupdate_scatter(hist_v, [idx], ones16)
                cur = lax.rem(c, NBUF)
                for b in range(NBUF):
                    @pl.when(cur == b)
                    def _():
                        pltpu.make_async_copy(y_hbm.at[src_v.at[c]],
                                              bufs[b], gsems[b]).wait()
                        pltpu.async_copy(bufs[b], acc_sh.at[dst_v.at[c]],
                                         ssems[b], add=True)
                        nb = (b + LAG) % NBUF

                        @pl.when(c + LAG < C)
                        def _():
                            @pl.when(c >= LAG)
                            def _():
                                pltpu.make_async_copy(
                                    bufs[nb], acc_sh.at[dst_v.at[c]],
                                    ssems[nb]).wait()

                            pltpu.async_copy(y_hbm.at[src_v.at[c + LAG]],
                                             bufs[nb], gsems[nb])
                return 0

            lax.fori_loop(0, C, step, 0)

            for j in range(max(0, C - NBUF), C):
                b = j % NBUF
                pltpu.make_async_copy(bufs[b], acc_sh.at[dst_v.at[j]],
                                      ssems[b]).wait()

        @pl.when(cid == 0)
        def _():
            run_pipeline(C0, sid * C0)

        @pl.when(cid == 1)
        def _():
            run_pipeline(C1, NS * C0 + sid * C1)

        plsc.subcore_barrier()

        pltpu.sync_copy(acc_sh.at[rows], out_hbm.at[cid, rows])
        if with_count:
            pltpu.sync_copy(hist_v, stage_sh.at[sid])
            plsc.subcore_barrier()
            pltpu.sync_copy(stage_sh.at[0, rows], csum_v)

            def accum(t, _):
                pltpu.sync_copy(stage_sh.at[t, rows], tmp_v)

                def addv(j, _):
                    sl = pl.ds(j * 16, 16)
                    csum_v[sl] = csum_v[sl] + tmp_v[sl]
                    return 0

                lax.fori_loop(0, R // 16, addv, 0)
                return 0

            lax.fori_loop(1, NS, accum, 0)

            @pl.when(cid == 0)
            def _():
                pltpu.sync_copy(csum_v, cnt0_hbm.at[rows])

            @pl.when(cid == 1)
            def _():
                pltpu.sync_copy(csum_v, cnt1_hbm.at[rows])

    return pl.kernel(body, out_type=out_type, mesh=mesh,
                     scratch_types=scratch,
                     compiler_params=pltpu.CompilerParams(
                         use_tc_tiling_on_sc=False,
                         needs_layout_passes=False))


def _mm2_body(x_ref, wl_ref, wr_ref, b_ref, y_ref, z_ref):
    x = x_ref[...]
    y = jnp.dot(x, wl_ref[...], preferred_element_type=jnp.float32)
    y_ref[...] = y.astype(jnp.bfloat16)
    z_ref[...] = (jnp.dot(x, wr_ref[...], preferred_element_type=jnp.float32)
                  + b_ref[...]).astype(jnp.bfloat16)


def _mean_relu(p_ref, c0_ref, c1_ref, z_ref):
    B = z_ref.shape[0]
    i = pl.program_id(0)
    off = pl.multiple_of(i * B, 128)
    cnt = c0_ref[pl.ds(off, B)] + c1_ref[pl.ds(off, B)]
    recip = 1.0 / jnp.maximum(cnt, 1.0).reshape(B, 1)
    psum = p_ref[0].astype(jnp.float32) + p_ref[1].astype(jnp.float32)
    return jnp.maximum(psum * recip + z_ref[...].astype(jnp.float32), 0.0)


def _mid_body(p_ref, c0_ref, c1_ref, z_ref, wl_ref, wr_ref, b_ref,
              y_ref, z2_ref):
    h = _mean_relu(p_ref, c0_ref, c1_ref, z_ref)
    y = jnp.dot(h, wl_ref[...], preferred_element_type=jnp.float32)
    y_ref[...] = y.astype(jnp.bfloat16)
    z2_ref[...] = (jnp.dot(h, wr_ref[...], preferred_element_type=jnp.float32)
                   + b_ref[...]).astype(jnp.bfloat16)


def _fin_body(p_ref, c0_ref, c1_ref, z_ref, out_ref):
    out_ref[...] = _mean_relu(p_ref, c0_ref, c1_ref, z_ref)


def _row_spec(B, D):
    return pl.BlockSpec((B, D), lambda i: (i, 0))


def _vec_spec(Np):
    return pl.BlockSpec((Np,), lambda i: (0,))


def _pair_spec(B, D):
    return pl.BlockSpec((NC, B, D), lambda i: (0, i, 0))


def _full_spec(shape):
    return pl.BlockSpec(shape, lambda i: tuple(0 for _ in shape))


@jax.jit
def kernel(x, edge_index, W1_l, W1_r, b1, W2_l, W2_r, b2):
    N, D = x.shape
    E = edge_index.shape[1]
    Np = ((N + NS * 16 - 1) // (NS * 16)) * NS * 16
    CT = -(-E // (NS * K)) * NS
    Ep = CT * K
    CPP = CT // NS
    C0 = (CPP * 645 + 500) // 1000
    C1 = CPP - C0
    B = 2048

    src = jnp.concatenate(
        [edge_index[0], jnp.zeros((Ep - E,), jnp.int32)])
    dst = jnp.concatenate(
        [edge_index[1], jnp.full((Ep - E,), N, jnp.int32)])
    src_r = src.reshape(CT, K)
    dst_r = dst.reshape(CT, K)
    zeros = jnp.zeros((Np, D), jnp.bfloat16)
    zeros1 = jnp.zeros((Np,), jnp.float32)
    b1r = b1.reshape(1, D)
    b2r = b2.reshape(1, D)

    grid = -(-N // B)
    mm2 = pl.pallas_call(
        _mm2_body, grid=(grid,),
        in_specs=[_row_spec(B, D), _full_spec((D, D)), _full_spec((D, D)),
                  _full_spec((1, D))],
        out_specs=[_row_spec(B, D), _row_spec(B, D)],
        out_shape=[jax.ShapeDtypeStruct((N, D), jnp.bfloat16),
                   jax.ShapeDtypeStruct((N, D), jnp.bfloat16)],
    )
    mid = pl.pallas_call(
        _mid_body, grid=(grid,),
        in_specs=[_pair_spec(B, D), _vec_spec(Np), _vec_spec(Np),
                  _row_spec(B, D),
                  _full_spec((D, D)), _full_spec((D, D)), _full_spec((1, D))],
        out_specs=[_row_spec(B, D), _row_spec(B, D)],
        out_shape=[jax.ShapeDtypeStruct((N, D), jnp.bfloat16),
                   jax.ShapeDtypeStruct((N, D), jnp.bfloat16)],
    )
    fin = pl.pallas_call(
        _fin_body, grid=(grid,),
        in_specs=[_pair_spec(B, D), _vec_spec(Np), _vec_spec(Np),
                  _row_spec(B, D)],
        out_specs=_row_spec(B, D),
        out_shape=jax.ShapeDtypeStruct((N, D), jnp.float32),
    )

    sc1 = _make_sc_agg(Np, D, C0, C1, with_count=True)
    sc2 = _make_sc_agg(Np, D, C0, C1, with_count=False)

    y1, z1 = mm2(x, W1_l, W1_r, b1r)
    p1, c0, c1 = sc1(src_r, dst_r, y1, zeros, zeros1)
    y2, z2 = mid(p1, c0, c1, z1, W2_l, W2_r, b2r)
    (p2,) = sc2(src_r, dst_r, y2, zeros)
    return fin(p2, c0, c1, z2)
